# Initial kernel scaffold; baseline (speedup 1.0000x reference)
import jax
import jax.numpy as jnp
from jax.experimental import pallas as pl


def kernel(entity, pattern, entity_emb, pat_emb, W1, b1, W2, b2):
    raise NotImplementedError("write your pallas kernel here")


# speedup vs baseline: 1.5910x; 1.5910x over previous
import functools

import jax
import jax.numpy as jnp
from jax import lax
from jax.experimental import pallas as pl
from jax.experimental.pallas import tpu as pltpu
from jax.experimental.pallas import tpu_sc as plsc

V = 1_000_000
D = 32
H = 64
O = 2
B = 16384
L = 20

NC, NS = 2, 16
NW = NC * NS
BPW = B // NW
CHUNK = 128
NCHUNK = BPW * L // CHUNK
SB = 32
CPS = SB * L // CHUNK
NSTAGE = BPW // SB
OP = 8


def _sc_pool(ent_i, pat_i, etbl, ptbl):
    mesh = plsc.VectorSubcoreMesh(core_axis_name="c", subcore_axis_name="s")

    @functools.partial(
        pl.kernel,
        mesh=mesh,
        compiler_params=pltpu.CompilerParams(use_tc_tiling_on_sc=False),
        out_type=jax.ShapeDtypeStruct((B, 2 * D), jnp.float32),
        scratch_types=[
            pltpu.VMEM((NCHUNK, CHUNK), jnp.int32),
            pltpu.VMEM((2, SB * L, D), jnp.float32),
            pltpu.VMEM((BPW, 2 * D), jnp.float32),
            pltpu.SemaphoreType.DMA,
            pltpu.SemaphoreType.DMA,
        ],
    )
    def k(ent_hbm, pat_hbm, et_hbm, pt_hbm, out_hbm, idx_v, buf, out_v, sem0, sem1):
        cid = lax.axis_index("c")
        sid = lax.axis_index("s")
        wid = sid * NC + cid
        base = wid * BPW
        inv = jnp.float32(1.0 / L)
        sems = (sem0, sem1)

        def run_table(idx_hbm, tbl, off):
            pltpu.sync_copy(idx_hbm.at[wid], idx_v)

            def fire(s, slot):
                for c in range(CPS):
                    pltpu.async_copy(
                        tbl.at[idx_v.at[s * CPS + c]],
                        buf.at[slot, pl.ds(c * CHUNK, CHUNK)],
                        sems[slot],
                    )

            def wait(slot):
                pltpu.make_async_copy(
                    tbl.at[pl.ds(0, SB * L)], buf.at[slot], sems[slot]
                ).wait()

            def compute(s, slot):
                sbase = s * SB

                def bbody(bb, _):
                    for kk in range(8):
                        b = bb * 8 + kk
                        r = b * L
                        for h0 in (0, 16):
                            parts = []
                            for p in range(4):
                                acc = buf[slot, r + p, h0:h0 + 16]
                                for l in range(p + 4, L, 4):
                                    acc = acc + buf[slot, r + l, h0:h0 + 16]
                                parts.append(acc)
                            tot = (parts[0] + parts[1]) + (parts[2] + parts[3])
                            out_v[sbase + b, off + h0:off + h0 + 16] = tot * inv
                    return 0

                lax.fori_loop(0, SB // 8, bbody, 0)

            fire(0, 0)

            def sbody(sp, _):
                s0 = 2 * sp
                fire(s0 + 1, 1)
                wait(0)
                compute(s0, 0)

                @pl.when(s0 + 2 < NSTAGE)
                def _():
                    fire(s0 + 2, 0)

                wait(1)
                compute(s0 + 1, 1)
                return 0

            lax.fori_loop(0, NSTAGE // 2, sbody, 0)

        run_table(ent_hbm, et_hbm, 0)
        run_table(pat_hbm, pt_hbm, D)
        pltpu.sync_copy(out_v, out_hbm.at[pl.ds(base, BPW)])

    return k(ent_i, pat_i, etbl, ptbl)


def _mlp_body(x_ref, w1_ref, b1_ref, w2_ref, b2_ref, o_ref):
    h = jnp.dot(x_ref[...], w1_ref[...], preferred_element_type=jnp.float32)
    h = jnp.maximum(h + b1_ref[...], 0.0)
    o = jnp.dot(h, w2_ref[...], preferred_element_type=jnp.float32)
    o_ref[...] = o + b2_ref[...]


def _mlp(x, w1t, b1r, w2t, b2r):
    blk = 2048
    grid = (B // blk,)
    return pl.pallas_call(
        _mlp_body,
        grid=grid,
        in_specs=[
            pl.BlockSpec((blk, 2 * D), lambda i: (i, 0)),
            pl.BlockSpec((2 * D, H), lambda i: (0, 0)),
            pl.BlockSpec((1, H), lambda i: (0, 0)),
            pl.BlockSpec((H, OP), lambda i: (0, 0)),
            pl.BlockSpec((1, OP), lambda i: (0, 0)),
        ],
        out_specs=pl.BlockSpec((blk, OP), lambda i: (i, 0)),
        out_shape=jax.ShapeDtypeStruct((B, OP), jnp.float32),
    )(x, w1t, b1r, w2t, b2r)


def kernel(entity, pattern, entity_emb, pat_emb, W1, b1, W2, b2):
    ent_i = entity.reshape(NW, NCHUNK, CHUNK)
    pat_i = pattern.reshape(NW, NCHUNK, CHUNK)
    pooled = _sc_pool(ent_i, pat_i, entity_emb, pat_emb)
    w1t = W1.T
    b1r = b1.reshape(1, H)
    w2t = jnp.zeros((H, OP), jnp.float32).at[:, :O].set(W2.T)
    b2r = jnp.zeros((1, OP), jnp.float32).at[:, :O].set(b2.reshape(1, O))
    out = _mlp(pooled, w1t, b1r, w2t, b2r)
    return out[:, :O]


# speedup vs baseline: 1.6366x; 1.0286x over previous
import functools

import jax
import jax.numpy as jnp
from jax import lax
from jax.experimental import pallas as pl
from jax.experimental.pallas import tpu as pltpu
from jax.experimental.pallas import tpu_sc as plsc

V = 1_000_000
D = 32
H = 64
O = 2
B = 16384
L = 20

NC, NS = 2, 16
NW = NC * NS
BPW = B // NW
CHUNK = 128
NCK = BPW // CHUNK
OP = 8


def _sc_pool(ent_i, pat_i, etbl, ptbl):
    mesh = plsc.VectorSubcoreMesh(core_axis_name="c", subcore_axis_name="s")

    @functools.partial(
        pl.kernel,
        mesh=mesh,
        compiler_params=pltpu.CompilerParams(use_tc_tiling_on_sc=False),
        out_type=jax.ShapeDtypeStruct((B, 2 * D), jnp.float32),
        scratch_types=[
            pltpu.VMEM((L, NCK, CHUNK), jnp.int32),
            pltpu.VMEM((L, NCK, CHUNK), jnp.int32),
            pltpu.VMEM((BPW, D), jnp.float32),
            pltpu.VMEM((BPW, D), jnp.float32),
            pltpu.VMEM((BPW, 2 * D), jnp.float32),
            pltpu.SemaphoreType.DMA,
            pltpu.SemaphoreType.DMA,
        ],
    )
    def k(ent_hbm, pat_hbm, et_hbm, pt_hbm, out_hbm,
          eidx_v, pidx_v, acce, accp, out_v, sem_i, sem_g):
        cid = lax.axis_index("c")
        sid = lax.axis_index("s")
        wid = sid * NC + cid
        base = wid * BPW
        inv = jnp.float32(1.0 / L)

        pltpu.async_copy(ent_hbm.at[:, wid], eidx_v, sem_i)
        pltpu.async_copy(pat_hbm.at[:, wid], pidx_v, sem_i)

        zero = jnp.zeros((16,), jnp.float32)

        def zbody(i, _):
            for j in range(4):
                for h0 in (0, 16):
                    acce[i * 4 + j, h0:h0 + 16] = zero
                    accp[i * 4 + j, h0:h0 + 16] = zero
            return 0

        lax.fori_loop(0, BPW // 4, zbody, 0)

        pltpu.make_async_copy(ent_hbm.at[:, wid], eidx_v, sem_i).wait()
        pltpu.make_async_copy(pat_hbm.at[:, wid], pidx_v, sem_i).wait()

        for l in range(L):
            for c in range(NCK):
                pltpu.async_copy(
                    et_hbm.at[eidx_v.at[l, c]],
                    acce.at[pl.ds(c * CHUNK, CHUNK)],
                    sem_g,
                    add=True,
                )
                pltpu.async_copy(
                    pt_hbm.at[pidx_v.at[l, c]],
                    accp.at[pl.ds(c * CHUNK, CHUNK)],
                    sem_g,
                    add=True,
                )

        for _ in range(L):
            pltpu.make_async_copy(et_hbm.at[pl.ds(0, BPW)], acce, sem_g).wait()
            pltpu.make_async_copy(pt_hbm.at[pl.ds(0, BPW)], accp, sem_g).wait()

        def sbody(i, _):
            for j in range(4):
                r = i * 4 + j
                for h0 in (0, 16):
                    out_v[r, h0:h0 + 16] = acce[r, h0:h0 + 16] * inv
                    out_v[r, D + h0:D + h0 + 16] = accp[r, h0:h0 + 16] * inv
            return 0

        lax.fori_loop(0, BPW // 4, sbody, 0)

        pltpu.sync_copy(out_v, out_hbm.at[pl.ds(base, BPW)])

    return k(ent_i, pat_i, etbl, ptbl)


def _mlp_body(x_ref, w1_ref, b1_ref, w2_ref, b2_ref, o_ref):
    h = jnp.dot(x_ref[...], w1_ref[...], preferred_element_type=jnp.float32)
    h = jnp.maximum(h + b1_ref[...], 0.0)
    o = jnp.dot(h, w2_ref[...], preferred_element_type=jnp.float32)
    o_ref[...] = o + b2_ref[...]


def _mlp(x, w1t, b1r, w2t, b2r):
    blk = 2048
    grid = (B // blk,)
    return pl.pallas_call(
        _mlp_body,
        grid=grid,
        in_specs=[
            pl.BlockSpec((blk, 2 * D), lambda i: (i, 0)),
            pl.BlockSpec((2 * D, H), lambda i: (0, 0)),
            pl.BlockSpec((1, H), lambda i: (0, 0)),
            pl.BlockSpec((H, OP), lambda i: (0, 0)),
            pl.BlockSpec((1, OP), lambda i: (0, 0)),
        ],
        out_specs=pl.BlockSpec((blk, OP), lambda i: (i, 0)),
        out_shape=jax.ShapeDtypeStruct((B, OP), jnp.float32),
    )(x, w1t, b1r, w2t, b2r)


def kernel(entity, pattern, entity_emb, pat_emb, W1, b1, W2, b2):
    ent_i = entity.T.reshape(L, NW, NCK, CHUNK)
    pat_i = pattern.T.reshape(L, NW, NCK, CHUNK)
    pooled = _sc_pool(ent_i, pat_i, entity_emb, pat_emb)
    w1t = W1.T
    b1r = b1.reshape(1, H)
    w2t = jnp.zeros((H, OP), jnp.float32).at[:, :O].set(W2.T)
    b2r = jnp.zeros((1, OP), jnp.float32).at[:, :O].set(b2.reshape(1, O))
    out = _mlp(pooled, w1t, b1r, w2t, b2r)
    return out[:, :O]
